# baseline (speedup 1.0000x reference)
import functools

import numpy as np
import jax
import jax.numpy as jnp
from jax import lax
from jax.experimental import pallas as pl
from jax.experimental.pallas import tpu as pltpu
from jax.experimental.pallas import tpu_sc as plsc

B, T, V, H = 4, 12, 256, 64
NPG = T * V
E = 98304
M = B * V * H
ROWW = B * H

NC, NS = 2, 16
EPC = E // NC
CH = 512

_F = np.fft.fft(np.eye(H), axis=0)
_CRT = np.ascontiguousarray(_F.real.T, dtype=np.float32)
_CIT = np.ascontiguousarray(_F.imag.T, dtype=np.float32)
_ART = np.ascontiguousarray(np.fft.irfft(np.eye(T), n=T, axis=-1).T, dtype=np.float32)
_AIT = np.ascontiguousarray(np.fft.irfft(1j * np.eye(T), n=T, axis=-1).T, dtype=np.float32)


def _dft_body(z_ref, crt_ref, cit_ref, zr_ref, zi_ref):
    z = z_ref[...]
    zr_ref[...] = jnp.dot(z, crt_ref[...], preferred_element_type=jnp.float32)
    zi_ref[...] = jnp.dot(z, cit_ref[...], preferred_element_type=jnp.float32)


def _dft(z):
    n = B * T * V
    g = 8
    blk = n // g
    return pl.pallas_call(
        _dft_body,
        grid=(g,),
        in_specs=[
            pl.BlockSpec((blk, H), lambda i: (i, 0)),
            pl.BlockSpec((H, H), lambda i: (0, 0)),
            pl.BlockSpec((H, H), lambda i: (0, 0)),
        ],
        out_specs=[
            pl.BlockSpec((blk, H), lambda i: (i, 0)),
            pl.BlockSpec((blk, H), lambda i: (i, 0)),
        ],
        out_shape=[
            jax.ShapeDtypeStruct((n, H), jnp.float32),
            jax.ShapeDtypeStruct((n, H), jnp.float32),
        ],
    )(z, _CRT, _CIT)


def _bd4(m):
    r, c = m.shape
    out = jnp.zeros((B * r, B * c), m.dtype)
    for i in range(B):
        out = out.at[i * r:(i + 1) * r, i * c:(i + 1) * c].set(m)
    return out


def _glu_body(xr_ref, xi_ref, art_ref, ait_ref, *wrefs):
    out_ref = wrefs[-1]
    wrefs = wrefs[:-1]
    streams = []
    for si, xref in enumerate((xr_ref, xi_ref)):
        x = xref[...]
        for li in range(3):
            base = si * 12 + li * 4
            wlt, bl, wrt, br = (wrefs[base + k][...] for k in range(4))
            left = jnp.dot(wlt, x, preferred_element_type=jnp.float32) + bl
            right = jnp.dot(wrt, x, preferred_element_type=jnp.float32) + br
            x = left * (1.0 / (1.0 + jnp.exp(-right)))
        streams.append(x)
    out_ref[...] = (jnp.dot(art_ref[...], streams[0], preferred_element_type=jnp.float32)
                    + jnp.dot(ait_ref[...], streams[1], preferred_element_type=jnp.float32))


def _glu_stage(xr, xi, params):
    warrs = []
    wspecs = []
    for stream in ("real", "img"):
        for p in params[stream]:
            for nm in ("wl", "wr"):
                wt = _bd4(p[nm].T)
                warrs.append(wt)
                wspecs.append(pl.BlockSpec(wt.shape, lambda i: (0, 0)))
                bt = jnp.tile(p["b" + nm[1]], B).reshape(-1, 1)
                warrs.append(bt)
                wspecs.append(pl.BlockSpec(bt.shape, lambda i: (0, 0)))
    vh = V * H
    g = 4
    nb = vh // g
    art4 = _bd4(jnp.asarray(_ART))
    ait4 = _bd4(jnp.asarray(_AIT))
    bt48 = B * T
    return pl.pallas_call(
        _glu_body,
        grid=(g,),
        in_specs=[
            pl.BlockSpec((bt48, nb), lambda i: (0, i)),
            pl.BlockSpec((bt48, nb), lambda i: (0, i)),
            pl.BlockSpec((bt48, bt48), lambda i: (0, 0)),
            pl.BlockSpec((bt48, bt48), lambda i: (0, 0)),
        ] + wspecs,
        out_specs=pl.BlockSpec((bt48, nb), lambda i: (0, i)),
        out_shape=jax.ShapeDtypeStruct((bt48, vh), jnp.float32),
    )(xr, xi, art4, ait4, *warrs)


def _graph_body(table_hbm, src_hbm, dst_hbm, w_hbm, out_hbm,
                tbl_v, src_s, dst_s, w_c, acc_v, src_sh, dst_sh, w_sh):
    c = lax.axis_index("c")
    s = lax.axis_index("s")
    b = s // (H // 16)
    hs = s % (H // 16)

    @pl.when(s == 0)
    def _():
        pltpu.sync_copy(src_hbm.at[pl.ds(c * EPC, EPC)], src_sh)
        pltpu.sync_copy(dst_hbm.at[pl.ds(c * EPC, EPC)], dst_sh)

    @pl.when(s == 1)
    def _():
        pltpu.sync_copy(w_hbm.at[pl.ds(c * EPC, EPC)], w_sh)

    pltpu.sync_copy(table_hbm.at[b, slice(None), pl.ds(hs * 16, 16)], tbl_v)
    plsc.subcore_barrier()

    def zrow(r, carry):
        acc_v[r, :] = jnp.zeros((16,), jnp.float32)
        return carry
    lax.fori_loop(0, NPG, zrow, 0)

    def chunk(k, carry):
        base = k * CH
        pltpu.sync_copy(src_sh.at[pl.ds(base, CH)], src_s)
        pltpu.sync_copy(dst_sh.at[pl.ds(base, CH)], dst_s)
        pltpu.sync_copy(w_sh.at[pl.ds(base, CH)], w_c)

        @plsc.parallel_loop(0, CH // 16, 1, unroll=2)
        def grp(g):
            w16 = w_c[pl.ds(g * 16, 16)]
            for i in range(16):
                e = g * 16 + i
                val = tbl_v[src_s[e], :]
                wspl = w16[jnp.full((16,), i, jnp.int32)]
                plsc.addupdate(acc_v.at[dst_s[e]], val * wspl)
        return carry
    lax.fori_loop(0, EPC // CH, chunk, 0)

    pltpu.sync_copy(acc_v, out_hbm.at[c, b, slice(None), pl.ds(hs * 16, 16)])


def _graph(table4, src1, dst1, w1):
    mesh = plsc.VectorSubcoreMesh(core_axis_name="c", subcore_axis_name="s")
    k = functools.partial(
        pl.kernel,
        mesh=mesh,
        compiler_params=pltpu.CompilerParams(
            use_tc_tiling_on_sc=False, needs_layout_passes=False),
        out_type=jax.ShapeDtypeStruct((NC, B, NPG, H), jnp.float32),
        scratch_types=[
            pltpu.VMEM((NPG, 16), jnp.float32),
            pltpu.SMEM((CH,), jnp.int32),
            pltpu.SMEM((CH,), jnp.int32),
            pltpu.VMEM((CH,), jnp.float32),
            pltpu.VMEM((NPG, 16), jnp.float32),
            pltpu.VMEM_SHARED((EPC,), jnp.int32),
            pltpu.VMEM_SHARED((EPC,), jnp.int32),
            pltpu.VMEM_SHARED((EPC,), jnp.float32),
        ],
    )(_graph_body)
    return k(table4, src1, dst1, w1)


def _gcn_body(acc_ref, w_ref, b_ref, out_ref):
    a = acc_ref[0] + acc_ref[1]
    o = jnp.dot(a, w_ref[...], preferred_element_type=jnp.float32) + b_ref[...]
    out_ref[...] = jnp.where(o >= 0, o, 0.01 * o)


def _gcn(acc, w, b):
    n = B * T * V // 2
    g = 8
    blk = n // g
    w2 = jnp.zeros((2 * H, 2 * H), jnp.float32)
    w2 = w2.at[:H, :H].set(w).at[H:, H:].set(w)
    b2 = jnp.tile(b, 2).reshape(1, 2 * H)
    return pl.pallas_call(
        _gcn_body,
        grid=(g,),
        in_specs=[
            pl.BlockSpec((2, blk, 2 * H), lambda i: (0, i, 0)),
            pl.BlockSpec((2 * H, 2 * H), lambda i: (0, 0)),
            pl.BlockSpec((1, 2 * H), lambda i: (0, 0)),
        ],
        out_specs=pl.BlockSpec((blk, 2 * H), lambda i: (i, 0)),
        out_shape=jax.ShapeDtypeStruct((n, 2 * H), jnp.float32),
    )(acc, w2, b2)


def kernel(x, params, edge_weight, edge_index):
    zr, zi = _dft(x.reshape(B * T * V, H))
    xr = zr.reshape(B * T, V * H)
    xi = zi.reshape(B * T, V * H)
    out_t = _glu_stage(xr, xi, params)
    table4 = out_t.reshape(B, NPG, H)
    src1 = edge_index[0]
    dst1 = edge_index[1]
    acc = _graph(table4, src1, dst1, edge_weight)
    accr = acc.reshape(NC, B * T * V // 2, 2 * H)
    out = _gcn(accr, params["gcn_w"], params["gcn_b"])
    return out.reshape(B, T, V, H)


# speedup vs baseline: 1.1048x; 1.1048x over previous
import functools

import numpy as np
import jax
import jax.numpy as jnp
from jax import lax
from jax.experimental import pallas as pl
from jax.experimental.pallas import tpu as pltpu
from jax.experimental.pallas import tpu_sc as plsc

B, T, V, H = 4, 12, 256, 64
NPG = T * V
E = 98304
M = B * V * H
ROWW = B * H

NC, NS = 2, 16
EPC = E // NC
CH = 512

_F = np.fft.fft(np.eye(H), axis=0)
_CRT = np.ascontiguousarray(_F.real.T, dtype=np.float32)
_CIT = np.ascontiguousarray(_F.imag.T, dtype=np.float32)
_ART = np.ascontiguousarray(np.fft.irfft(np.eye(T), n=T, axis=-1).T, dtype=np.float32)
_AIT = np.ascontiguousarray(np.fft.irfft(1j * np.eye(T), n=T, axis=-1).T, dtype=np.float32)


def _dft_body(z_ref, crt_ref, cit_ref, zr_ref, zi_ref):
    z = z_ref[...]
    zr_ref[...] = jnp.dot(z, crt_ref[...], preferred_element_type=jnp.float32)
    zi_ref[...] = jnp.dot(z, cit_ref[...], preferred_element_type=jnp.float32)


def _dft(z):
    n = B * T * V
    g = 8
    blk = n // g
    return pl.pallas_call(
        _dft_body,
        grid=(g,),
        in_specs=[
            pl.BlockSpec((blk, H), lambda i: (i, 0)),
            pl.BlockSpec((H, H), lambda i: (0, 0)),
            pl.BlockSpec((H, H), lambda i: (0, 0)),
        ],
        out_specs=[
            pl.BlockSpec((blk, H), lambda i: (i, 0)),
            pl.BlockSpec((blk, H), lambda i: (i, 0)),
        ],
        out_shape=[
            jax.ShapeDtypeStruct((n, H), jnp.float32),
            jax.ShapeDtypeStruct((n, H), jnp.float32),
        ],
    )(z, _CRT, _CIT)


def _bd4(m):
    return jnp.kron(jnp.eye(B, dtype=m.dtype), m)


def _glu_body(xr_ref, xi_ref, art_ref, ait_ref, *wrefs):
    out_ref = wrefs[-1]
    wrefs = wrefs[:-1]
    streams = []
    for si, xref in enumerate((xr_ref, xi_ref)):
        x = xref[...]
        for li in range(3):
            base = si * 12 + li * 4
            wlt, bl, wrt, br = (wrefs[base + k][...] for k in range(4))
            left = jnp.dot(wlt, x, preferred_element_type=jnp.float32) + bl
            right = jnp.dot(wrt, x, preferred_element_type=jnp.float32) + br
            x = left * (1.0 / (1.0 + jnp.exp(-right)))
        streams.append(x)
    out_ref[...] = (jnp.dot(art_ref[...], streams[0], preferred_element_type=jnp.float32)
                    + jnp.dot(ait_ref[...], streams[1], preferred_element_type=jnp.float32))


def _glu_stage(xr, xi, params):
    warrs = []
    wspecs = []
    for stream in ("real", "img"):
        for p in params[stream]:
            for nm in ("wl", "wr"):
                wt = _bd4(p[nm].T)
                warrs.append(wt)
                wspecs.append(pl.BlockSpec(wt.shape, lambda i: (0, 0)))
                bt = jnp.tile(p["b" + nm[1]], B).reshape(-1, 1)
                warrs.append(bt)
                wspecs.append(pl.BlockSpec(bt.shape, lambda i: (0, 0)))
    vh = V * H
    g = 4
    nb = vh // g
    art4 = np.kron(np.eye(B, dtype=np.float32), _ART)
    ait4 = np.kron(np.eye(B, dtype=np.float32), _AIT)
    bt48 = B * T
    return pl.pallas_call(
        _glu_body,
        grid=(g,),
        in_specs=[
            pl.BlockSpec((bt48, nb), lambda i: (0, i)),
            pl.BlockSpec((bt48, nb), lambda i: (0, i)),
            pl.BlockSpec((bt48, bt48), lambda i: (0, 0)),
            pl.BlockSpec((bt48, bt48), lambda i: (0, 0)),
        ] + wspecs,
        out_specs=pl.BlockSpec((bt48, nb), lambda i: (0, i)),
        out_shape=jax.ShapeDtypeStruct((bt48, vh), jnp.float32),
    )(xr, xi, art4, ait4, *warrs)


def _graph_body(table_hbm, src_hbm, dst_hbm, w_hbm, out_hbm,
                tbl_v, src_s, dst_s, w_c, acc_v, src_sh, dst_sh, w_sh):
    c = lax.axis_index("c")
    s = lax.axis_index("s")
    b = s // (H // 16)
    hs = s % (H // 16)

    @pl.when(s == 0)
    def _():
        pltpu.sync_copy(src_hbm.at[pl.ds(c * EPC, EPC)], src_sh)
        pltpu.sync_copy(dst_hbm.at[pl.ds(c * EPC, EPC)], dst_sh)

    @pl.when(s == 1)
    def _():
        pltpu.sync_copy(w_hbm.at[pl.ds(c * EPC, EPC)], w_sh)

    pltpu.sync_copy(table_hbm.at[b, slice(None), pl.ds(hs * 16, 16)], tbl_v)
    plsc.subcore_barrier()

    def zrow(r, carry):
        acc_v[r, :] = jnp.zeros((16,), jnp.float32)
        return carry
    lax.fori_loop(0, NPG, zrow, 0)

    def chunk(k, carry):
        base = k * CH
        pltpu.sync_copy(src_sh.at[pl.ds(base, CH)], src_s)
        pltpu.sync_copy(dst_sh.at[pl.ds(base, CH)], dst_s)
        pltpu.sync_copy(w_sh.at[pl.ds(base, CH)], w_c)

        @plsc.parallel_loop(0, CH // 16, 1, unroll=2)
        def grp(g):
            w16 = w_c[pl.ds(g * 16, 16)]
            for i in range(16):
                e = g * 16 + i
                val = tbl_v[src_s[e], :]
                wspl = w16[jnp.full((16,), i, jnp.int32)]
                plsc.addupdate(acc_v.at[dst_s[e]], val * wspl)
        return carry
    lax.fori_loop(0, EPC // CH, chunk, 0)

    pltpu.sync_copy(acc_v, out_hbm.at[c, b, slice(None), pl.ds(hs * 16, 16)])


def _graph(table4, src1, dst1, w1):
    mesh = plsc.VectorSubcoreMesh(core_axis_name="c", subcore_axis_name="s")
    k = functools.partial(
        pl.kernel,
        mesh=mesh,
        compiler_params=pltpu.CompilerParams(
            use_tc_tiling_on_sc=False, needs_layout_passes=False),
        out_type=jax.ShapeDtypeStruct((NC, B, NPG, H), jnp.float32),
        scratch_types=[
            pltpu.VMEM((NPG, 16), jnp.float32),
            pltpu.SMEM((CH,), jnp.int32),
            pltpu.SMEM((CH,), jnp.int32),
            pltpu.VMEM((CH,), jnp.float32),
            pltpu.VMEM((NPG, 16), jnp.float32),
            pltpu.VMEM_SHARED((EPC,), jnp.int32),
            pltpu.VMEM_SHARED((EPC,), jnp.int32),
            pltpu.VMEM_SHARED((EPC,), jnp.float32),
        ],
    )(_graph_body)
    return k(table4, src1, dst1, w1)


def _gcn_body(acc_ref, w_ref, b_ref, out_ref):
    a = acc_ref[0] + acc_ref[1]
    o = jnp.dot(a, w_ref[...], preferred_element_type=jnp.float32) + b_ref[...]
    out_ref[...] = jnp.where(o >= 0, o, 0.01 * o)


def _gcn(acc, w, b):
    n = B * T * V // 2
    g = 8
    blk = n // g
    w2 = jnp.kron(jnp.eye(2, dtype=jnp.float32), w)
    b2 = jnp.tile(b, 2).reshape(1, 2 * H)
    return pl.pallas_call(
        _gcn_body,
        grid=(g,),
        in_specs=[
            pl.BlockSpec((2, blk, 2 * H), lambda i: (0, i, 0)),
            pl.BlockSpec((2 * H, 2 * H), lambda i: (0, 0)),
            pl.BlockSpec((1, 2 * H), lambda i: (0, 0)),
        ],
        out_specs=pl.BlockSpec((blk, 2 * H), lambda i: (i, 0)),
        out_shape=jax.ShapeDtypeStruct((n, 2 * H), jnp.float32),
    )(acc, w2, b2)


def kernel(x, params, edge_weight, edge_index):
    zr, zi = _dft(x.reshape(B * T * V, H))
    xr = zr.reshape(B * T, V * H)
    xi = zi.reshape(B * T, V * H)
    out_t = _glu_stage(xr, xi, params)
    table4 = out_t.reshape(B, NPG, H)
    src1 = edge_index[0]
    dst1 = edge_index[1]
    acc = _graph(table4, src1, dst1, edge_weight)
    accr = acc.reshape(NC, B * T * V // 2, 2 * H)
    out = _gcn(accr, params["gcn_w"], params["gcn_b"])
    return out.reshape(B, T, V, H)
